# baseline (speedup 1.0000x reference)
import functools

import jax
import jax.numpy as jnp
from jax import lax
from jax.experimental import pallas as pl
from jax.experimental.pallas import tpu as pltpu
from jax.experimental.pallas import tpu_sc as plsc

_BATCH = 16384
_DIM = 4096
_OPS = 16

_NC = 2
_NS = 16
_NW = _NC * _NS
_ROWS_PER_W = _BATCH // _NW
_RCHUNK = 4
_NCHUNKS = _ROWS_PER_W // _RCHUNK
_NJ = _DIM // 16

_GATE_COEF = (
    (0.0, 0.0, 0.0, 0.0),
    (0.0, 0.0, 0.0, 1.0),
    (0.0, 1.0, 0.0, -1.0),
    (0.0, 1.0, 0.0, 0.0),
    (0.0, 0.0, 1.0, -1.0),
    (0.0, 0.0, 1.0, 0.0),
    (0.0, 1.0, 1.0, -2.0),
    (0.0, 1.0, 1.0, -1.0),
    (1.0, -1.0, -1.0, 1.0),
    (1.0, -1.0, -1.0, 2.0),
    (1.0, 0.0, -1.0, 0.0),
    (1.0, 0.0, -1.0, 1.0),
    (1.0, -1.0, 0.0, 0.0),
    (1.0, -1.0, 0.0, 1.0),
    (1.0, 0.0, 0.0, -1.0),
    (1.0, 0.0, 0.0, 0.0),
)


def _sc_kernel(x_hbm, wt_hbm, ia_hbm, ib_hbm, out_hbm,
               ia_v, ib_v, c0_v, ca_v, cb_v, cab_v, w_v, xbuf, obuf):
    wid = lax.axis_index("s") * _NC + lax.axis_index("c")
    base_row = wid * _ROWS_PER_W

    pltpu.sync_copy(ia_hbm, ia_v)
    pltpu.sync_copy(ib_hbm, ib_v)
    pltpu.sync_copy(wt_hbm, w_v)

    @pl.loop(0, _NJ)
    def _coef(j):
        off = j * 16
        w = [w_v[i, pl.ds(off, 16)] for i in range(_OPS)]
        m = w[0]
        for i in range(1, _OPS):
            m = jnp.maximum(m, w[i])
        e = [jnp.exp(w[i] - m) for i in range(_OPS)]
        s = e[0]
        for i in range(1, _OPS):
            s = s + e[i]
        inv = 1.0 / s
        acc = [None, None, None, None]
        for i in range(_OPS):
            for k in range(4):
                g = _GATE_COEF[i][k]
                if g != 0.0:
                    t = e[i] if g == 1.0 else e[i] * g
                    acc[k] = t if acc[k] is None else acc[k] + t
        c0_v[pl.ds(off, 16)] = acc[0] * inv
        ca_v[pl.ds(off, 16)] = acc[1] * inv
        cb_v[pl.ds(off, 16)] = acc[2] * inv
        cab_v[pl.ds(off, 16)] = acc[3] * inv

    @pl.loop(0, _NCHUNKS)
    def _chunk(g):
        row0 = base_row + g * _RCHUNK
        pltpu.sync_copy(x_hbm.at[pl.ds(row0, _RCHUNK), :], xbuf)

        @pl.loop(0, _NJ)
        def _blk(j):
            off = j * 16
            ia16 = ia_v[pl.ds(off, 16)]
            ib16 = ib_v[pl.ds(off, 16)]
            c0 = c0_v[pl.ds(off, 16)]
            ca = ca_v[pl.ds(off, 16)]
            cb = cb_v[pl.ds(off, 16)]
            cab = cab_v[pl.ds(off, 16)]
            for r in range(_RCHUNK):
                rvec = jnp.full((16,), r, dtype=jnp.int32)
                a = plsc.load_gather(xbuf, [rvec, ia16])
                b = plsc.load_gather(xbuf, [rvec, ib16])
                obuf[r, pl.ds(off, 16)] = c0 + a * (ca + cab * b) + cb * b

        pltpu.sync_copy(obuf, out_hbm.at[pl.ds(row0, _RCHUNK), :])


@jax.jit
def _run(x, wt, ia, ib):
    mesh = plsc.VectorSubcoreMesh(core_axis_name="c", subcore_axis_name="s")
    f = pl.kernel(
        _sc_kernel,
        out_type=jax.ShapeDtypeStruct((_BATCH, _DIM), jnp.float32),
        mesh=mesh,
        compiler_params=pltpu.CompilerParams(needs_layout_passes=False),
        scratch_types=[
            pltpu.VMEM((_DIM,), jnp.int32),
            pltpu.VMEM((_DIM,), jnp.int32),
            pltpu.VMEM((_DIM,), jnp.float32),
            pltpu.VMEM((_DIM,), jnp.float32),
            pltpu.VMEM((_DIM,), jnp.float32),
            pltpu.VMEM((_DIM,), jnp.float32),
            pltpu.VMEM((_OPS, _DIM), jnp.float32),
            pltpu.VMEM((_RCHUNK, _DIM), jnp.float32),
            pltpu.VMEM((_RCHUNK, _DIM), jnp.float32),
        ],
    )
    return f(x, wt, ia, ib)


def kernel(x, weights, indices_a, indices_b):
    wt = jnp.transpose(weights)
    return _run(x, wt, indices_a, indices_b)


# speedup vs baseline: 9.3253x; 9.3253x over previous
import functools

import jax
import jax.numpy as jnp
from jax import lax
from jax.experimental import pallas as pl
from jax.experimental.pallas import tpu as pltpu
from jax.experimental.pallas import tpu_sc as plsc

_BATCH = 16384
_DIM = 4096
_OPS = 16

_NC = 2
_NS = 16
_NW = _NC * _NS
_ROWS_PER_W = _BATCH // _NW
_RCHUNK = 4
_NCHUNKS = _ROWS_PER_W // _RCHUNK
_NJ = _DIM // 16

_GATE_COEF = (
    (0.0, 0.0, 0.0, 0.0),
    (0.0, 0.0, 0.0, 1.0),
    (0.0, 1.0, 0.0, -1.0),
    (0.0, 1.0, 0.0, 0.0),
    (0.0, 0.0, 1.0, -1.0),
    (0.0, 0.0, 1.0, 0.0),
    (0.0, 1.0, 1.0, -2.0),
    (0.0, 1.0, 1.0, -1.0),
    (1.0, -1.0, -1.0, 1.0),
    (1.0, -1.0, -1.0, 2.0),
    (1.0, 0.0, -1.0, 0.0),
    (1.0, 0.0, -1.0, 1.0),
    (1.0, -1.0, 0.0, 0.0),
    (1.0, -1.0, 0.0, 1.0),
    (1.0, 0.0, 0.0, -1.0),
    (1.0, 0.0, 0.0, 0.0),
)


def _coef_tc(wt_ref, g_ref, out_ref):
    w = wt_ref[...]
    m = jnp.max(w, axis=0, keepdims=True)
    e = jnp.exp(w - m)
    e = e / jnp.sum(e, axis=0, keepdims=True)
    out_ref[...] = jnp.dot(g_ref[...], e, preferred_element_type=jnp.float32)


def _sc_kernel(x_hbm, cf_hbm, ia_hbm, ib_hbm, out_hbm,
               ia_v, ib_v, cf_v,
               xbuf0, xbuf1, obuf0, obuf1,
               sem_i0, sem_i1, sem_o0, sem_o1):
    wid = lax.axis_index("s") * _NC + lax.axis_index("c")
    base_row = wid * _ROWS_PER_W
    xbufs = (xbuf0, xbuf1)
    obufs = (obuf0, obuf1)
    sem_in = (sem_i0, sem_i1)
    sem_out = (sem_o0, sem_o1)

    pltpu.sync_copy(ia_hbm, ia_v)
    pltpu.sync_copy(ib_hbm, ib_v)
    pltpu.sync_copy(cf_hbm, cf_v)

    for b in range(2):
        row0 = base_row + b * _RCHUNK
        pltpu.async_copy(x_hbm.at[pl.ds(row0, _RCHUNK), :], xbufs[b], sem_in[b])

    @pl.loop(0, _NCHUNKS, step=2)
    def _chunk(g):
        for b in range(2):
            c = g + b
            row0 = base_row + c * _RCHUNK
            xb, ob = xbufs[b], obufs[b]
            pltpu.make_async_copy(
                x_hbm.at[pl.ds(row0, _RCHUNK), :], xb, sem_in[b]).wait()

            @pl.when(g > 0)
            def _():
                prow0 = row0 - 2 * _RCHUNK
                pltpu.make_async_copy(
                    ob, out_hbm.at[pl.ds(prow0, _RCHUNK), :], sem_out[b]).wait()

            @functools.partial(plsc.parallel_loop, 0, _NJ, unroll=4)
            def _blk(j):
                off = j * 16
                ia16 = ia_v[pl.ds(off, 16)]
                ib16 = ib_v[pl.ds(off, 16)]
                c0 = cf_v[0, pl.ds(off, 16)]
                ca = cf_v[1, pl.ds(off, 16)]
                cb = cf_v[2, pl.ds(off, 16)]
                cab = cf_v[3, pl.ds(off, 16)]
                for r in range(_RCHUNK):
                    rvec = jnp.full((16,), r, dtype=jnp.int32)
                    a = plsc.load_gather(xb, [rvec, ia16])
                    bb = plsc.load_gather(xb, [rvec, ib16])
                    ob[r, pl.ds(off, 16)] = c0 + a * (ca + cab * bb) + cb * bb

            pltpu.async_copy(ob, out_hbm.at[pl.ds(row0, _RCHUNK), :], sem_out[b])

            @pl.when(c + 2 < _NCHUNKS)
            def _():
                nrow0 = row0 + 2 * _RCHUNK
                pltpu.async_copy(
                    x_hbm.at[pl.ds(nrow0, _RCHUNK), :], xb, sem_in[b])

    for b in range(2):
        row0 = base_row + (_NCHUNKS - 2 + b) * _RCHUNK
        pltpu.make_async_copy(
            obufs[b], out_hbm.at[pl.ds(row0, _RCHUNK), :], sem_out[b]).wait()


@jax.jit
def _run(x, wt, g, ia, ib):
    coef = pl.pallas_call(
        _coef_tc,
        out_shape=jax.ShapeDtypeStruct((4, _DIM), jnp.float32),
    )(wt, g)

    mesh = plsc.VectorSubcoreMesh(core_axis_name="c", subcore_axis_name="s")
    f = pl.kernel(
        _sc_kernel,
        out_type=jax.ShapeDtypeStruct((_BATCH, _DIM), jnp.float32),
        mesh=mesh,
        compiler_params=pltpu.CompilerParams(needs_layout_passes=False),
        scratch_types=[
            pltpu.VMEM((_DIM,), jnp.int32),
            pltpu.VMEM((_DIM,), jnp.int32),
            pltpu.VMEM((4, _DIM), jnp.float32),
            pltpu.VMEM((_RCHUNK, _DIM), jnp.float32),
            pltpu.VMEM((_RCHUNK, _DIM), jnp.float32),
            pltpu.VMEM((_RCHUNK, _DIM), jnp.float32),
            pltpu.VMEM((_RCHUNK, _DIM), jnp.float32),
            pltpu.SemaphoreType.DMA,
            pltpu.SemaphoreType.DMA,
            pltpu.SemaphoreType.DMA,
            pltpu.SemaphoreType.DMA,
        ],
    )
    return f(x, coef, ia, ib)


def kernel(x, weights, indices_a, indices_b):
    wt = jnp.transpose(weights)
    g = jnp.transpose(jnp.asarray(_GATE_COEF, dtype=jnp.float32))
    return _run(x, wt, g, indices_a, indices_b)
